# Initial kernel scaffold; baseline (speedup 1.0000x reference)
import jax
import jax.numpy as jnp
from jax.experimental import pallas as pl


def kernel(x, edge_index, W1, b1, W2, b2):
    raise NotImplementedError("write your pallas kernel here")


# speedup vs baseline: 157.9414x; 157.9414x over previous
import functools

import jax
import jax.numpy as jnp
from jax import lax
from jax.experimental import pallas as pl
from jax.experimental.pallas import tpu as pltpu
from jax.experimental.pallas import tpu_sc as plsc

N = 100000
N_PAD = 100352
R = N_PAD // 128
E = 6400000
NC, NS = 2, 16
NTILES = NC * NS
K = 16
ROWS_PER_TILE = 1568
NBLK = ROWS_PER_TILE // K
TOT_ROWS = NTILES * ROWS_PER_TILE
E_PAD = TOT_ROWS * 128

_MESH = plsc.VectorSubcoreMesh(core_axis_name="c", subcore_axis_name="s")
_F32 = jnp.float32


def _sds(shape):
    return jax.ShapeDtypeStruct(shape, _F32)


@functools.partial(
    pl.kernel,
    out_type=_sds((NC, N_PAD)),
    mesh=_MESH,
    scratch_types=[
        pltpu.VMEM((K, 128), jnp.int32),
        pltpu.VMEM((K, 128), _F32),
        pltpu.VMEM_SHARED((N_PAD,), _F32),
        pltpu.SemaphoreType.DMA,
    ],
)
def _deg_pass(dst_hbm, ones_hbm, zeros_hbm, out_hbm, dst_v, ones_v, s_sh, sem):
    cid = lax.axis_index("c")
    sid = lax.axis_index("s")
    tile = cid * NS + sid

    @pl.when(sid == 0)
    def _stage():
        pltpu.sync_copy(zeros_hbm, s_sh)

    pltpu.sync_copy(ones_hbm, ones_v)
    plsc.subcore_barrier()

    base = tile * ROWS_PER_TILE

    def blk(b, carry):
        r0 = base + b * K
        pltpu.sync_copy(dst_hbm.at[pl.ds(r0, K)], dst_v)
        descs = [
            pltpu.async_copy(ones_v.at[j], s_sh.at[dst_v.at[j]], sem, add=True)
            for j in range(K)
        ]
        for d in descs:
            d.wait()
        return carry

    lax.fori_loop(0, NBLK, blk, 0)

    plsc.subcore_barrier()

    @pl.when(sid == 0)
    def _flush():
        pltpu.sync_copy(s_sh, out_hbm.at[cid])


def _make_prop(ncols):
    scratch = [
        pltpu.VMEM((K, 128), jnp.int32),
        pltpu.VMEM((K, 128), jnp.int32),
    ]
    for _ in range(ncols):
        scratch.append(pltpu.VMEM((K, 128), _F32))
    for _ in range(ncols):
        scratch.append(pltpu.VMEM_SHARED((N_PAD,), _F32))
    for _ in range(ncols):
        scratch.append(pltpu.VMEM_SHARED((N_PAD,), _F32))
    scratch += [pltpu.SemaphoreType.DMA, pltpu.SemaphoreType.DMA]

    out_type = [_sds((NC, N_PAD)) for _ in range(ncols)]

    @functools.partial(pl.kernel, out_type=out_type, mesh=_MESH,
                       scratch_types=scratch)
    def _prop(*refs):
        src_hbm = refs[0]
        dst_hbm = refs[1]
        u_hbm = refs[2:2 + ncols]
        zeros_hbm = refs[2 + ncols]
        outs = refs[3 + ncols:3 + 2 * ncols]
        pos = 3 + 2 * ncols
        src_v, dst_v = refs[pos], refs[pos + 1]
        vals = refs[pos + 2:pos + 2 + ncols]
        u_sh = refs[pos + 2 + ncols:pos + 2 + 2 * ncols]
        s_sh = refs[pos + 2 + 2 * ncols:pos + 2 + 3 * ncols]
        gsem, ssem = refs[pos + 2 + 3 * ncols], refs[pos + 3 + 3 * ncols]

        cid = lax.axis_index("c")
        sid = lax.axis_index("s")
        tile = cid * NS + sid

        @pl.when(sid == 0)
        def _stage():
            for c in range(ncols):
                pltpu.sync_copy(u_hbm[c], u_sh[c])
                pltpu.sync_copy(zeros_hbm, s_sh[c])

        plsc.subcore_barrier()

        base = tile * ROWS_PER_TILE

        def blk(b, carry):
            r0 = base + b * K
            pltpu.sync_copy(src_hbm.at[pl.ds(r0, K)], src_v)
            pltpu.sync_copy(dst_hbm.at[pl.ds(r0, K)], dst_v)
            gd = [
                pltpu.async_copy(u_sh[c].at[src_v.at[j]], vals[c].at[j], gsem)
                for c in range(ncols)
                for j in range(K)
            ]
            for d in gd:
                d.wait()
            sd = [
                pltpu.async_copy(vals[c].at[j], s_sh[c].at[dst_v.at[j]], ssem,
                                 add=True)
                for c in range(ncols)
                for j in range(K)
            ]
            for d in sd:
                d.wait()
            return carry

        lax.fori_loop(0, NBLK, blk, 0)

        plsc.subcore_barrier()

        @pl.when(sid == 0)
        def _flush():
            for c in range(ncols):
                pltpu.sync_copy(s_sh[c], outs[c].at[cid])

    return _prop


_prop1 = _make_prop(1)
_prop2 = _make_prop(2)


def _node_mask():
    row = lax.broadcasted_iota(jnp.int32, (R, 128), 0)
    col = lax.broadcasted_iota(jnp.int32, (R, 128), 1)
    return (row * 128 + col) < N


def _stage_a(dega, degb, x0):
    def body(dega_ref, degb_ref, x0_ref, d_ref, u0_ref):
        deg = dega_ref[...] + degb_ref[...] + 1.0
        d = lax.rsqrt(deg)
        d_ref[...] = d
        u0_ref[...] = d * x0_ref[...]

    return pl.pallas_call(
        body, out_shape=(_sds((R, 128)), _sds((R, 128))),
    )(dega, degb, x0)


def _stage_b(sa, sb, u0, d, W1, b1, W2):
    def body(sa_ref, sb_ref, u0_ref, d_ref, w1_ref, b1_ref, w2_ref,
             u1_ref, u2_ref):
        d = d_ref[...]
        p = d * (sa_ref[...] + sb_ref[...] + u0_ref[...])
        z0 = jnp.zeros((R, 128), _F32)
        z1 = jnp.zeros((R, 128), _F32)
        for j in range(16):
            h = jnp.maximum(p * w1_ref[0, j] + b1_ref[j], 0.0)
            z0 = z0 + h * w2_ref[j, 0]
            z1 = z1 + h * w2_ref[j, 1]
        mask = _node_mask()
        u1_ref[...] = jnp.where(mask, d * z0, 0.0)
        u2_ref[...] = jnp.where(mask, d * z1, 0.0)

    vm = pl.BlockSpec(memory_space=pltpu.VMEM)
    sm = pl.BlockSpec(memory_space=pltpu.SMEM)
    return pl.pallas_call(
        body,
        in_specs=[vm, vm, vm, vm, sm, sm, sm],
        out_specs=(vm, vm),
        out_shape=(_sds((R, 128)), _sds((R, 128))),
    )(sa, sb, u0, d, W1, b1, W2)


def _stage_c(s0a, s0b, s1a, s1b, u1, u2, d, b2):
    def body(s0a_ref, s0b_ref, s1a_ref, s1b_ref, u1_ref, u2_ref, d_ref,
             b2_ref, y_ref):
        d = d_ref[...]
        o0 = d * (s0a_ref[...] + s0b_ref[...] + u1_ref[...]) + b2_ref[0]
        o1 = d * (s1a_ref[...] + s1b_ref[...] + u2_ref[...]) + b2_ref[1]
        m = jnp.maximum(o0, o1)
        lse = m + jnp.log(jnp.exp(o0 - m) + jnp.exp(o1 - m))
        y_ref[0] = o0 - lse
        y_ref[1] = o1 - lse

    vm = pl.BlockSpec(memory_space=pltpu.VMEM)
    sm = pl.BlockSpec(memory_space=pltpu.SMEM)
    return pl.pallas_call(
        body,
        in_specs=[vm, vm, vm, vm, vm, vm, vm, sm],
        out_specs=vm,
        out_shape=_sds((2, R, 128)),
    )(s0a, s0b, s1a, s1b, u1, u2, d, b2)


def kernel(x, edge_index, W1, b1, W2, b2):
    src = edge_index[0].astype(jnp.int32)
    dst = edge_index[1].astype(jnp.int32)
    pad = jnp.full((E_PAD - E,), N, jnp.int32)
    src2d = jnp.concatenate([src, pad]).reshape(TOT_ROWS, 128)
    dst2d = jnp.concatenate([dst, pad]).reshape(TOT_ROWS, 128)

    x0 = jnp.pad(x[:, 0].astype(_F32), (0, N_PAD - N))
    zeros = jnp.zeros((N_PAD,), _F32)
    ones_blk = jnp.ones((K, 128), _F32)

    deg = _deg_pass(dst2d, ones_blk, zeros)
    deg2 = deg.reshape(NC, R, 128)
    d, u0 = _stage_a(deg2[0], deg2[1], x0.reshape(R, 128))

    s0 = _prop1(src2d, dst2d, u0.reshape(N_PAD), zeros)
    if isinstance(s0, (list, tuple)):
        s0 = s0[0]
    s0 = s0.reshape(NC, R, 128)

    u1, u2 = _stage_b(s0[0], s0[1], u0, d,
                      W1.astype(_F32), b1.astype(_F32), W2.astype(_F32))

    t0, t1 = _prop2(src2d, dst2d, u1.reshape(N_PAD), u2.reshape(N_PAD), zeros)
    t0 = t0.reshape(NC, R, 128)
    t1 = t1.reshape(NC, R, 128)

    y = _stage_c(t0[0], t0[1], t1[0], t1[1], u1, u2, d, b2.astype(_F32))
    return y.reshape(2, N_PAD)[:, :N].T
